# baseline (speedup 1.0000x reference)
import jax
import jax.numpy as jnp
from jax import lax
from jax.experimental import pallas as pl
from jax.experimental.pallas import tpu as pltpu
from jax.experimental.pallas import tpu_sc as plsc

N = 10000
E = 320000
D = 128
DH = 64
NB = 16

NPAD = 10240
RB = 512
GRID = NPAD // RB
NC = 2
NS = 16
G = 128
E_TOT = E + N
NGRP = 162
E_PAD = NS * NGRP * G
ROWS_PER_TILE = NPAD // NS
EPS = 1e-16


def _split_store(h_ref, h):
    h_ref[0] = h[:, :DH]
    h_ref[1] = h[:, DH:]


def _tc_prep_body(x_ref, w_ref, a_ref, h_ref, sc_ref):
    h = jnp.dot(x_ref[...], w_ref[...], preferred_element_type=jnp.float32)
    _split_store(h_ref, h)
    sc_ref[...] = jnp.dot(h, a_ref[...], preferred_element_type=jnp.float32)


def _tc_prep(x_pad, W, A):
    return pl.pallas_call(
        _tc_prep_body,
        grid=(GRID,),
        in_specs=[
            pl.BlockSpec((RB, D), lambda i: (i, 0)),
            pl.BlockSpec((D, D), lambda i: (0, 0)),
            pl.BlockSpec((D, 8), lambda i: (0, 0)),
        ],
        out_specs=[
            pl.BlockSpec((NC, RB, DH), lambda i: (0, i, 0)),
            pl.BlockSpec((RB, 8), lambda i: (i, 0)),
        ],
        out_shape=[
            jax.ShapeDtypeStruct((NC, NPAD, DH), jnp.float32),
            jax.ShapeDtypeStruct((NPAD, 8), jnp.float32),
        ],
    )(x_pad, W, A)


def _norm(u_ref, den_ref, b_ref):
    u = jnp.concatenate([u_ref[0], u_ref[1]], axis=1)
    den = jnp.sum(den_ref[0], axis=0) + EPS
    return u / den[:, None] + b_ref[0:1, :]


def _tc_mid_body(u_ref, den_ref, b_ref, w_ref, a_ref, h_ref, sc_ref):
    hmid = jax.nn.relu(_norm(u_ref, den_ref, b_ref))
    h2 = jnp.dot(hmid, w_ref[...], preferred_element_type=jnp.float32)
    _split_store(h_ref, h2)
    sc_ref[...] = jnp.dot(h2, a_ref[...], preferred_element_type=jnp.float32)


def _tc_mid(U, den, b_r, W, A):
    return pl.pallas_call(
        _tc_mid_body,
        grid=(GRID,),
        in_specs=[
            pl.BlockSpec((NC, RB, DH), lambda i: (0, i, 0)),
            pl.BlockSpec((NC, NS, RB), lambda i: (0, 0, i)),
            pl.BlockSpec((8, D), lambda i: (0, 0)),
            pl.BlockSpec((D, D), lambda i: (0, 0)),
            pl.BlockSpec((D, 8), lambda i: (0, 0)),
        ],
        out_specs=[
            pl.BlockSpec((NC, RB, DH), lambda i: (0, i, 0)),
            pl.BlockSpec((RB, 8), lambda i: (i, 0)),
        ],
        out_shape=[
            jax.ShapeDtypeStruct((NC, NPAD, DH), jnp.float32),
            jax.ShapeDtypeStruct((NPAD, 8), jnp.float32),
        ],
    )(U, den, b_r, W, A)


def _tc_pool_body(u_ref, den_ref, b_ref, batch_ref, out_ref, acc_s, acc_c):
    i = pl.program_id(0)

    @pl.when(i == 0)
    def _():
        acc_s[...] = jnp.zeros_like(acc_s)
        acc_c[...] = jnp.zeros_like(acc_c)

    hf = _norm(u_ref, den_ref, b_ref)
    bt = batch_ref[0, 0, :]
    ids = lax.broadcasted_iota(jnp.int32, (NB, RB), 0)
    oh = (bt[None, :] == ids).astype(jnp.float32)
    acc_s[...] += jnp.dot(oh, hf, preferred_element_type=jnp.float32)
    acc_c[...] += jnp.sum(oh, axis=1, keepdims=True)

    @pl.when(i == GRID - 1)
    def _():
        out_ref[...] = acc_s[...] / jnp.clip(acc_c[...], 1.0, None)


def _tc_pool(U, den, b_r, batch3):
    return pl.pallas_call(
        _tc_pool_body,
        grid=(GRID,),
        in_specs=[
            pl.BlockSpec((NC, RB, DH), lambda i: (0, i, 0)),
            pl.BlockSpec((NC, NS, RB), lambda i: (0, 0, i)),
            pl.BlockSpec((8, D), lambda i: (0, 0)),
            pl.BlockSpec((1, 1, RB), lambda i: (i, 0, 0)),
        ],
        out_specs=pl.BlockSpec((NB, D), lambda i: (0, 0)),
        out_shape=jax.ShapeDtypeStruct((NB, D), jnp.float32),
        scratch_shapes=[
            pltpu.VMEM((NB, D), jnp.float32),
            pltpu.VMEM((NB, 1), jnp.float32),
        ],
    )(U, den, b_r, batch3)


def _sc_edge_body(src_hbm, dst_hbm, as_hbm, ad_hbm, h_hbm,
                  u_out, den_out,
                  src_v, dst_v, as_v, ad_v, den_v,
                  w0_v, w1_v, rows0_v, rows1_v, u_sh,
                  gsem0, gsem1, ssem0, ssem1):
    c = lax.axis_index("c")
    s = lax.axis_index("s")
    base = s * ROWS_PER_TILE
    w_bufs = (w0_v, w1_v)
    rows_bufs = (rows0_v, rows1_v)
    gsems = (gsem0, gsem1)
    ssems = (ssem0, ssem1)

    pltpu.sync_copy(src_hbm.at[s], src_v)
    pltpu.sync_copy(dst_hbm.at[s], dst_v)
    pltpu.sync_copy(as_hbm, as_v)
    pltpu.sync_copy(ad_hbm, ad_v)

    zero16 = jnp.zeros((16,), jnp.float32)

    def zden(i, _):
        den_v[pl.ds(i * 16, 16)] = zero16
        return 0

    lax.fori_loop(0, NPAD // 16, zden, 0)

    def zrows(i, _):
        for j in range(DH // 16):
            rows0_v[i, pl.ds(j * 16, 16)] = zero16
        return 0

    lax.fori_loop(0, G, zrows, 0)

    def zu(k, _):
        pltpu.sync_copy(rows0_v, u_sh.at[pl.ds(base + k * G, G)])
        return 0

    lax.fori_loop(0, ROWS_PER_TILE // G, zu, 0)
    plsc.subcore_barrier()

    def compute_w(g, wbuf):
        for i in range(G // 16):
            s16 = src_v[g, pl.ds(i * 16, 16)]
            d16 = dst_v[g, pl.ds(i * 16, 16)]
            e = plsc.load_gather(as_v, [s16]) + plsc.load_gather(ad_v, [d16])
            e = jnp.where(e >= 0.0, e, e * 0.2)
            w16 = jnp.exp(e)
            plsc.addupdate_scatter(den_v, [d16], w16)
            wbuf[pl.ds(i * 16, 16)] = w16

    def issue_gather(g, b):
        pltpu.async_copy(h_hbm.at[c].at[src_v.at[g]], rows_bufs[b], gsems[b])

    def wait_gather(g, b):
        pltpu.make_async_copy(
            h_hbm.at[c].at[src_v.at[g]], rows_bufs[b], gsems[b]
        ).wait()

    def scale(b):
        wbuf = w_bufs[b]
        rbuf = rows_bufs[b]

        def srow(i4, _):
            for u in range(4):
                i = i4 * 4 + u
                wb = plsc.load_gather(wbuf, [jnp.full((16,), i, jnp.int32)])
                for j in range(DH // 16):
                    rbuf[i, pl.ds(j * 16, 16)] = rbuf[i, pl.ds(j * 16, 16)] * wb
            return 0

        lax.fori_loop(0, G // 4, srow, 0)

    def drain_scatter(b):
        pltpu.make_async_copy(
            rows_bufs[b], u_sh.at[pl.ds(0, G)], ssems[b]
        ).wait()

    def body(g, b, wait_prev, prefetch):
        wait_gather(g, b)
        scale(b)
        pltpu.async_copy(rows_bufs[b], u_sh.at[dst_v.at[g]], ssems[b], add=True)
        if prefetch:
            if wait_prev:
                drain_scatter(1 - b)
            issue_gather(g + 1, 1 - b)
            compute_w(g + 1, w_bufs[1 - b])

    issue_gather(0, 0)
    compute_w(0, w0_v)
    body(0, 0, wait_prev=False, prefetch=True)

    def pair(k, _):
        g = 1 + k * 2
        body(g, 1, wait_prev=True, prefetch=True)
        body(g + 1, 0, wait_prev=True, prefetch=True)
        return 0

    lax.fori_loop(0, (NGRP - 2) // 2, pair, 0)
    body(NGRP - 1, 1, wait_prev=True, prefetch=False)
    drain_scatter(0)
    drain_scatter(1)
    plsc.subcore_barrier()

    def cu(k, _):
        r0 = base + k * G
        pltpu.sync_copy(u_sh.at[pl.ds(r0, G)], u_out.at[c, pl.ds(r0, G)])
        return 0

    lax.fori_loop(0, ROWS_PER_TILE // G, cu, 0)
    pltpu.sync_copy(den_v, den_out.at[c, s])


_sc_edge = pl.kernel(
    _sc_edge_body,
    out_type=(
        jax.ShapeDtypeStruct((NC, NPAD, DH), jnp.float32),
        jax.ShapeDtypeStruct((NC, NS, NPAD), jnp.float32),
    ),
    mesh=plsc.VectorSubcoreMesh(
        core_axis_name="c", subcore_axis_name="s", num_cores=NC, num_subcores=NS
    ),
    compiler_params=pltpu.CompilerParams(
        needs_layout_passes=False, use_tc_tiling_on_sc=False
    ),
    scratch_types=[
        pltpu.VMEM((NGRP, G), jnp.int32),
        pltpu.VMEM((NGRP, G), jnp.int32),
        pltpu.VMEM((NPAD,), jnp.float32),
        pltpu.VMEM((NPAD,), jnp.float32),
        pltpu.VMEM((NPAD,), jnp.float32),
        pltpu.VMEM((G,), jnp.float32),
        pltpu.VMEM((G,), jnp.float32),
        pltpu.VMEM((G, DH), jnp.float32),
        pltpu.VMEM((G, DH), jnp.float32),
        pltpu.VMEM_SHARED((NPAD, DH), jnp.float32),
        pltpu.SemaphoreType.DMA,
        pltpu.SemaphoreType.DMA,
        pltpu.SemaphoreType.DMA,
        pltpu.SemaphoreType.DMA,
    ],
)


def kernel(x, edge_index, batch, W1, a1_src, a1_dst, b1, W2, a2_src, a2_dst, b2):
    loop = jnp.arange(N, dtype=jnp.int32)
    pad = E_PAD - E_TOT
    src = jnp.concatenate([edge_index[0], loop, jnp.zeros((pad,), jnp.int32)])
    dst = jnp.concatenate([edge_index[1], loop, jnp.full((pad,), N, jnp.int32)])
    src3 = src.reshape(NS, NGRP, G)
    dst3 = dst.reshape(NS, NGRP, G)

    x_pad = jnp.concatenate([x, jnp.zeros((NPAD - N, D), jnp.float32)])
    A1 = jnp.zeros((D, 8), jnp.float32).at[:, 0].set(a1_src).at[:, 1].set(a1_dst)
    A2 = jnp.zeros((D, 8), jnp.float32).at[:, 0].set(a2_src).at[:, 1].set(a2_dst)
    b1r = jnp.zeros((8, D), jnp.float32).at[0].set(b1)
    b2r = jnp.zeros((8, D), jnp.float32).at[0].set(b2)
    batch3 = jnp.concatenate(
        [batch.astype(jnp.int32), jnp.full((NPAD - N,), NB, jnp.int32)]
    ).reshape(GRID, 1, RB)

    h1, sc1 = _tc_prep(x_pad, W1, A1)
    U1, den1 = _sc_edge(src3, dst3, sc1[:, 0], sc1[:, 1], h1)
    h2, sc2 = _tc_mid(U1, den1, b1r, W2, A2)
    U2, den2 = _sc_edge(src3, dst3, sc2[:, 0], sc2[:, 1], h2)
    return _tc_pool(U2, den2, b2r, batch3)


# speedup vs baseline: 1.2597x; 1.2597x over previous
import jax
import jax.numpy as jnp
from jax import lax
from jax.experimental import pallas as pl
from jax.experimental.pallas import tpu as pltpu
from jax.experimental.pallas import tpu_sc as plsc

N = 10000
E = 320000
D = 128
DH = 64
NB = 16

NPAD = 10240
RB = 512
GRID = NPAD // RB
NC = 2
NS = 16
G = 128
E_TOT = E + N
NGRP = 162
NBUF = 2
E_PAD = NS * NGRP * G
ROWS_PER_TILE = NPAD // NS
EPS = 1e-16


def _split_store(h_ref, h):
    h_ref[0] = h[:, :DH]
    h_ref[1] = h[:, DH:]


def _tc_prep_body(x_ref, w_ref, a_ref, h_ref, sc_ref):
    h = jnp.dot(x_ref[...], w_ref[...], preferred_element_type=jnp.float32)
    _split_store(h_ref, h)
    sc_ref[...] = jnp.dot(h, a_ref[...], preferred_element_type=jnp.float32)


def _tc_prep(x_pad, W, A):
    return pl.pallas_call(
        _tc_prep_body,
        grid=(GRID,),
        in_specs=[
            pl.BlockSpec((RB, D), lambda i: (i, 0)),
            pl.BlockSpec((D, D), lambda i: (0, 0)),
            pl.BlockSpec((D, 8), lambda i: (0, 0)),
        ],
        out_specs=[
            pl.BlockSpec((NC, RB, DH), lambda i: (0, i, 0)),
            pl.BlockSpec((RB, 8), lambda i: (i, 0)),
        ],
        out_shape=[
            jax.ShapeDtypeStruct((NC, NPAD, DH), jnp.float32),
            jax.ShapeDtypeStruct((NPAD, 8), jnp.float32),
        ],
    )(x_pad, W, A)


def _norm(u_ref, den_ref, b_ref):
    u = jnp.concatenate([u_ref[0], u_ref[1]], axis=1)
    den = jnp.sum(den_ref[0], axis=0) + EPS
    return u / den[:, None] + b_ref[0:1, :]


def _tc_mid_body(u_ref, den_ref, b_ref, w_ref, a_ref, h_ref, sc_ref):
    hmid = jax.nn.relu(_norm(u_ref, den_ref, b_ref))
    h2 = jnp.dot(hmid, w_ref[...], preferred_element_type=jnp.float32)
    _split_store(h_ref, h2)
    sc_ref[...] = jnp.dot(h2, a_ref[...], preferred_element_type=jnp.float32)


def _tc_mid(U, den, b_r, W, A):
    return pl.pallas_call(
        _tc_mid_body,
        grid=(GRID,),
        in_specs=[
            pl.BlockSpec((NC, RB, DH), lambda i: (0, i, 0)),
            pl.BlockSpec((NC, NS, RB), lambda i: (0, 0, i)),
            pl.BlockSpec((8, D), lambda i: (0, 0)),
            pl.BlockSpec((D, D), lambda i: (0, 0)),
            pl.BlockSpec((D, 8), lambda i: (0, 0)),
        ],
        out_specs=[
            pl.BlockSpec((NC, RB, DH), lambda i: (0, i, 0)),
            pl.BlockSpec((RB, 8), lambda i: (i, 0)),
        ],
        out_shape=[
            jax.ShapeDtypeStruct((NC, NPAD, DH), jnp.float32),
            jax.ShapeDtypeStruct((NPAD, 8), jnp.float32),
        ],
    )(U, den, b_r, W, A)


def _tc_pool_body(u_ref, den_ref, b_ref, batch_ref, out_ref, acc_s, acc_c):
    i = pl.program_id(0)

    @pl.when(i == 0)
    def _():
        acc_s[...] = jnp.zeros_like(acc_s)
        acc_c[...] = jnp.zeros_like(acc_c)

    hf = _norm(u_ref, den_ref, b_ref)
    bt = batch_ref[0, 0, :]
    ids = lax.broadcasted_iota(jnp.int32, (NB, RB), 0)
    oh = (bt[None, :] == ids).astype(jnp.float32)
    acc_s[...] += jnp.dot(oh, hf, preferred_element_type=jnp.float32)
    acc_c[...] += jnp.sum(oh, axis=1, keepdims=True)

    @pl.when(i == GRID - 1)
    def _():
        out_ref[...] = acc_s[...] / jnp.clip(acc_c[...], 1.0, None)


def _tc_pool(U, den, b_r, batch3):
    return pl.pallas_call(
        _tc_pool_body,
        grid=(GRID,),
        in_specs=[
            pl.BlockSpec((NC, RB, DH), lambda i: (0, i, 0)),
            pl.BlockSpec((NC, NS, RB), lambda i: (0, 0, i)),
            pl.BlockSpec((8, D), lambda i: (0, 0)),
            pl.BlockSpec((1, 1, RB), lambda i: (i, 0, 0)),
        ],
        out_specs=pl.BlockSpec((NB, D), lambda i: (0, 0)),
        out_shape=jax.ShapeDtypeStruct((NB, D), jnp.float32),
        scratch_shapes=[
            pltpu.VMEM((NB, D), jnp.float32),
            pltpu.VMEM((NB, 1), jnp.float32),
        ],
    )(U, den, b_r, batch3)


def _sc_edge_body(src_hbm, dst_hbm, as_hbm, ad_hbm, h_hbm,
                  u_out, den_out,
                  src_v, dst_v, as_v, ad_v, den_v,
                  w0_v, w1_v, rows0_v, rows1_v, u_sh,
                  gsem0, gsem1, ssem0, ssem1):
    c = lax.axis_index("c")
    s = lax.axis_index("s")
    base = s * ROWS_PER_TILE
    w_bufs = (w0_v, w1_v)
    rows_bufs = (rows0_v, rows1_v)
    gsems = (gsem0, gsem1)
    ssems = (ssem0, ssem1)

    pltpu.sync_copy(src_hbm.at[s], src_v)
    pltpu.sync_copy(dst_hbm.at[s], dst_v)
    pltpu.sync_copy(as_hbm, as_v)
    pltpu.sync_copy(ad_hbm, ad_v)

    zero16 = jnp.zeros((16,), jnp.float32)

    def zden(i, _):
        den_v[pl.ds(i * 16, 16)] = zero16
        return 0

    lax.fori_loop(0, NPAD // 16, zden, 0)

    def zrows(i, _):
        for j in range(DH // 16):
            rows0_v[i, pl.ds(j * 16, 16)] = zero16
        return 0

    lax.fori_loop(0, G, zrows, 0)

    def zu(k, _):
        pltpu.sync_copy(
            rows0_v.at[pl.ds(0, 64)], u_sh.at[pl.ds(base + k * 64, 64)]
        )
        return 0

    lax.fori_loop(0, ROWS_PER_TILE // 64, zu, 0)
    plsc.subcore_barrier()

    def compute_w(g, wbuf):
        for i in range(G // 16):
            s16 = src_v[g, pl.ds(i * 16, 16)]
            d16 = dst_v[g, pl.ds(i * 16, 16)]
            e = plsc.load_gather(as_v, [s16]) + plsc.load_gather(ad_v, [d16])
            e = jnp.where(e >= 0.0, e, e * 0.2)
            w16 = jnp.exp(e)
            plsc.addupdate_scatter(den_v, [d16], w16)
            wbuf[pl.ds(i * 16, 16)] = w16

    def issue_gather(g, b):
        pltpu.async_copy(h_hbm.at[c].at[src_v.at[g]], rows_bufs[b], gsems[b])

    def wait_gather(g, b):
        pltpu.make_async_copy(
            h_hbm.at[c].at[src_v.at[g]], rows_bufs[b], gsems[b]
        ).wait()

    def scale(b):
        wbuf = w_bufs[b]
        rbuf = rows_bufs[b]

        def srow(i4, _):
            for u in range(4):
                i = i4 * 4 + u
                wb = plsc.load_gather(wbuf, [jnp.full((16,), i, jnp.int32)])
                for j in range(DH // 16):
                    rbuf[i, pl.ds(j * 16, 16)] = rbuf[i, pl.ds(j * 16, 16)] * wb
            return 0

        lax.fori_loop(0, G // 4, srow, 0)

    def drain_scatter(b):
        pltpu.make_async_copy(
            rows_bufs[b], u_sh.at[pl.ds(0, G)], ssems[b]
        ).wait()

    def body(g, b, drain, prefetch):
        nb = (b + NBUF - 1) % NBUF
        wait_gather(g, b)
        if prefetch:
            if drain:
                drain_scatter(nb)
            issue_gather(g + NBUF - 1, nb)
        scale(b)
        pltpu.async_copy(rows_bufs[b], u_sh.at[dst_v.at[g]], ssems[b], add=True)
        if prefetch:
            compute_w(g + NBUF - 1, w_bufs[nb])

    for g0 in range(NBUF - 1):
        issue_gather(g0, g0)
        compute_w(g0, w_bufs[g0])
    body(0, 0, drain=False, prefetch=True)
    for g0 in range(1, NBUF):
        body(g0, g0, drain=True, prefetch=True)

    def ring(k, _):
        for j in range(NBUF):
            body(k * NBUF + j, j, drain=True, prefetch=True)
        return 0

    lax.fori_loop(1, NGRP // NBUF - 1, ring, 0)
    body(NGRP - NBUF, 0, drain=True, prefetch=True)
    for g0 in range(NGRP - NBUF + 1, NGRP):
        body(g0, g0 % NBUF, drain=False, prefetch=False)
    for b in range(NBUF):
        drain_scatter(b)
    plsc.subcore_barrier()

    def cu(k, _):
        r0 = base + k * 64
        pltpu.sync_copy(u_sh.at[pl.ds(r0, 64)], u_out.at[c, pl.ds(r0, 64)])
        return 0

    lax.fori_loop(0, ROWS_PER_TILE // 64, cu, 0)
    pltpu.sync_copy(den_v, den_out.at[c, s])


_sc_edge = pl.kernel(
    _sc_edge_body,
    out_type=(
        jax.ShapeDtypeStruct((NC, NPAD, DH), jnp.float32),
        jax.ShapeDtypeStruct((NC, NS, NPAD), jnp.float32),
    ),
    mesh=plsc.VectorSubcoreMesh(
        core_axis_name="c", subcore_axis_name="s", num_cores=NC, num_subcores=NS
    ),
    compiler_params=pltpu.CompilerParams(
        needs_layout_passes=False, use_tc_tiling_on_sc=False
    ),
    scratch_types=[
        pltpu.VMEM((NGRP, G), jnp.int32),
        pltpu.VMEM((NGRP, G), jnp.int32),
        pltpu.VMEM((NPAD,), jnp.float32),
        pltpu.VMEM((NPAD,), jnp.float32),
        pltpu.VMEM((NPAD,), jnp.float32),
        pltpu.VMEM((G,), jnp.float32),
        pltpu.VMEM((G,), jnp.float32),
        pltpu.VMEM((G, DH), jnp.float32),
        pltpu.VMEM((G, DH), jnp.float32),
        pltpu.VMEM_SHARED((NPAD, DH), jnp.float32),
        pltpu.SemaphoreType.DMA,
        pltpu.SemaphoreType.DMA,
        pltpu.SemaphoreType.DMA,
        pltpu.SemaphoreType.DMA,
    ],
)


def kernel(x, edge_index, batch, W1, a1_src, a1_dst, b1, W2, a2_src, a2_dst, b2):
    loop = jnp.arange(N, dtype=jnp.int32)
    pad = E_PAD - E_TOT
    src = jnp.concatenate([edge_index[0], loop, jnp.zeros((pad,), jnp.int32)])
    dst = jnp.concatenate([edge_index[1], loop, jnp.full((pad,), N, jnp.int32)])
    src3 = src.reshape(NS, NGRP, G)
    dst3 = dst.reshape(NS, NGRP, G)

    x_pad = jnp.concatenate([x, jnp.zeros((NPAD - N, D), jnp.float32)])
    A1 = jnp.zeros((D, 8), jnp.float32).at[:, 0].set(a1_src).at[:, 1].set(a1_dst)
    A2 = jnp.zeros((D, 8), jnp.float32).at[:, 0].set(a2_src).at[:, 1].set(a2_dst)
    b1r = jnp.zeros((8, D), jnp.float32).at[0].set(b1)
    b2r = jnp.zeros((8, D), jnp.float32).at[0].set(b2)
    batch3 = jnp.concatenate(
        [batch.astype(jnp.int32), jnp.full((NPAD - N,), NB, jnp.int32)]
    ).reshape(GRID, 1, RB)

    h1, sc1 = _tc_prep(x_pad, W1, A1)
    U1, den1 = _sc_edge(src3, dst3, sc1[:, 0], sc1[:, 1], h1)
    h2, sc2 = _tc_mid(U1, den1, b1r, W2, A2)
    U2, den2 = _sc_edge(src3, dst3, sc2[:, 0], sc2[:, 1], h2)
    return _tc_pool(U2, den2, b2r, batch3)


# speedup vs baseline: 1.3758x; 1.0922x over previous
import jax
import jax.numpy as jnp
from jax import lax
from jax.experimental import pallas as pl
from jax.experimental.pallas import tpu as pltpu
from jax.experimental.pallas import tpu_sc as plsc

N = 10000
E = 320000
D = 128
DH = 64
NB = 16

NPAD = 10240
RB = 512
GRID = NPAD // RB
NC = 2
NS = 16
G = 80
E_TOT = E + N
NGRP = 258
NBUF = 3
E_PAD = NS * NGRP * G
ROWS_PER_TILE = NPAD // NS
EPS = 1e-16


def _split_store(h_ref, h):
    h_ref[0] = h[:, :DH]
    h_ref[1] = h[:, DH:]


def _tc_prep_body(x_ref, w_ref, a_ref, h_ref, sc_ref):
    h = jnp.dot(x_ref[...], w_ref[...], preferred_element_type=jnp.float32)
    _split_store(h_ref, h)
    sc_ref[...] = jnp.dot(h, a_ref[...], preferred_element_type=jnp.float32)


def _tc_prep(x_pad, W, A):
    return pl.pallas_call(
        _tc_prep_body,
        grid=(GRID,),
        in_specs=[
            pl.BlockSpec((RB, D), lambda i: (i, 0)),
            pl.BlockSpec((D, D), lambda i: (0, 0)),
            pl.BlockSpec((D, 8), lambda i: (0, 0)),
        ],
        out_specs=[
            pl.BlockSpec((NC, RB, DH), lambda i: (0, i, 0)),
            pl.BlockSpec((RB, 8), lambda i: (i, 0)),
        ],
        out_shape=[
            jax.ShapeDtypeStruct((NC, NPAD, DH), jnp.float32),
            jax.ShapeDtypeStruct((NPAD, 8), jnp.float32),
        ],
    )(x_pad, W, A)


def _norm(u_ref, den_ref, b_ref):
    u = jnp.concatenate([u_ref[0], u_ref[1]], axis=1)
    den = jnp.sum(den_ref[0], axis=0) + EPS
    return u / den[:, None] + b_ref[0:1, :]


def _tc_mid_body(u_ref, den_ref, b_ref, w_ref, a_ref, h_ref, sc_ref):
    hmid = jax.nn.relu(_norm(u_ref, den_ref, b_ref))
    h2 = jnp.dot(hmid, w_ref[...], preferred_element_type=jnp.float32)
    _split_store(h_ref, h2)
    sc_ref[...] = jnp.dot(h2, a_ref[...], preferred_element_type=jnp.float32)


def _tc_mid(U, den, b_r, W, A):
    return pl.pallas_call(
        _tc_mid_body,
        grid=(GRID,),
        in_specs=[
            pl.BlockSpec((NC, RB, DH), lambda i: (0, i, 0)),
            pl.BlockSpec((NC, NS, RB), lambda i: (0, 0, i)),
            pl.BlockSpec((8, D), lambda i: (0, 0)),
            pl.BlockSpec((D, D), lambda i: (0, 0)),
            pl.BlockSpec((D, 8), lambda i: (0, 0)),
        ],
        out_specs=[
            pl.BlockSpec((NC, RB, DH), lambda i: (0, i, 0)),
            pl.BlockSpec((RB, 8), lambda i: (i, 0)),
        ],
        out_shape=[
            jax.ShapeDtypeStruct((NC, NPAD, DH), jnp.float32),
            jax.ShapeDtypeStruct((NPAD, 8), jnp.float32),
        ],
    )(U, den, b_r, W, A)


def _tc_pool_body(u_ref, den_ref, b_ref, batch_ref, out_ref, acc_s, acc_c):
    i = pl.program_id(0)

    @pl.when(i == 0)
    def _():
        acc_s[...] = jnp.zeros_like(acc_s)
        acc_c[...] = jnp.zeros_like(acc_c)

    hf = _norm(u_ref, den_ref, b_ref)
    bt = batch_ref[0, 0, :]
    ids = lax.broadcasted_iota(jnp.int32, (NB, RB), 0)
    oh = (bt[None, :] == ids).astype(jnp.float32)
    acc_s[...] += jnp.dot(oh, hf, preferred_element_type=jnp.float32)
    acc_c[...] += jnp.sum(oh, axis=1, keepdims=True)

    @pl.when(i == GRID - 1)
    def _():
        out_ref[...] = acc_s[...] / jnp.clip(acc_c[...], 1.0, None)


def _tc_pool(U, den, b_r, batch3):
    return pl.pallas_call(
        _tc_pool_body,
        grid=(GRID,),
        in_specs=[
            pl.BlockSpec((NC, RB, DH), lambda i: (0, i, 0)),
            pl.BlockSpec((NC, NS, RB), lambda i: (0, 0, i)),
            pl.BlockSpec((8, D), lambda i: (0, 0)),
            pl.BlockSpec((1, 1, RB), lambda i: (i, 0, 0)),
        ],
        out_specs=pl.BlockSpec((NB, D), lambda i: (0, 0)),
        out_shape=jax.ShapeDtypeStruct((NB, D), jnp.float32),
        scratch_shapes=[
            pltpu.VMEM((NB, D), jnp.float32),
            pltpu.VMEM((NB, 1), jnp.float32),
        ],
    )(U, den, b_r, batch3)


def _sc_edge_body(src_hbm, dst_hbm, as_hbm, ad_hbm, h_hbm,
                  u_out, den_out,
                  src_v, dst_v, as_v, ad_v, den_v,
                  w0_v, w1_v, w2_v, rows0_v, rows1_v, rows2_v, u_sh,
                  gsem0, gsem1, gsem2, ssem0, ssem1, ssem2):
    c = lax.axis_index("c")
    s = lax.axis_index("s")
    base = s * ROWS_PER_TILE
    w_bufs = (w0_v, w1_v, w2_v)
    rows_bufs = (rows0_v, rows1_v, rows2_v)
    gsems = (gsem0, gsem1, gsem2)
    ssems = (ssem0, ssem1, ssem2)

    pltpu.sync_copy(src_hbm.at[s], src_v)
    pltpu.sync_copy(dst_hbm.at[s], dst_v)
    pltpu.sync_copy(as_hbm, as_v)
    pltpu.sync_copy(ad_hbm, ad_v)

    zero16 = jnp.zeros((16,), jnp.float32)

    def zden(i, _):
        den_v[pl.ds(i * 16, 16)] = zero16
        return 0

    lax.fori_loop(0, NPAD // 16, zden, 0)

    def zrows(i, _):
        for j in range(DH // 16):
            rows0_v[i, pl.ds(j * 16, 16)] = zero16
        return 0

    lax.fori_loop(0, G, zrows, 0)

    def zu(k, _):
        pltpu.sync_copy(
            rows0_v.at[pl.ds(0, 64)], u_sh.at[pl.ds(base + k * 64, 64)]
        )
        return 0

    lax.fori_loop(0, ROWS_PER_TILE // 64, zu, 0)
    plsc.subcore_barrier()

    def compute_w(g, wbuf):
        for i in range(G // 16):
            s16 = src_v[g, pl.ds(i * 16, 16)]
            d16 = dst_v[g, pl.ds(i * 16, 16)]
            e = plsc.load_gather(as_v, [s16]) + plsc.load_gather(ad_v, [d16])
            e = jnp.where(e >= 0.0, e, e * 0.2)
            w16 = jnp.exp(e)
            plsc.addupdate_scatter(den_v, [d16], w16)
            wbuf[pl.ds(i * 16, 16)] = w16

    def issue_gather(g, b):
        pltpu.async_copy(h_hbm.at[c].at[src_v.at[g]], rows_bufs[b], gsems[b])

    def wait_gather(g, b):
        pltpu.make_async_copy(
            h_hbm.at[c].at[src_v.at[g]], rows_bufs[b], gsems[b]
        ).wait()

    def scale(b):
        wbuf = w_bufs[b]
        rbuf = rows_bufs[b]

        def srow(i4, _):
            for u in range(4):
                i = i4 * 4 + u
                wb = plsc.load_gather(wbuf, [jnp.full((16,), i, jnp.int32)])
                for j in range(DH // 16):
                    rbuf[i, pl.ds(j * 16, 16)] = rbuf[i, pl.ds(j * 16, 16)] * wb
            return 0

        lax.fori_loop(0, G // 4, srow, 0)

    def drain_scatter(b):
        pltpu.make_async_copy(
            rows_bufs[b], u_sh.at[pl.ds(0, G)], ssems[b]
        ).wait()

    def body(g, b):
        nb = (b + NBUF - 1) % NBUF
        wait_gather(g, b)

        @pl.when(jnp.logical_and(g >= 1, g <= NGRP - NBUF))
        def _():
            drain_scatter(nb)

        @pl.when(g <= NGRP - NBUF)
        def _():
            issue_gather(g + NBUF - 1, nb)

        scale(b)
        pltpu.async_copy(rows_bufs[b], u_sh.at[dst_v.at[g]], ssems[b], add=True)

        @pl.when(g <= NGRP - NBUF)
        def _():
            compute_w(g + NBUF - 1, w_bufs[nb])

    for g0 in range(NBUF - 1):
        issue_gather(g0, g0)
        compute_w(g0, w_bufs[g0])

    def ring(k, _):
        for j in range(NBUF):
            body(k * NBUF + j, j)
        return 0

    lax.fori_loop(0, NGRP // NBUF, ring, 0)
    for b in range(NBUF):
        drain_scatter(b)
    plsc.subcore_barrier()

    def cu(k, _):
        r0 = base + k * 64
        pltpu.sync_copy(u_sh.at[pl.ds(r0, 64)], u_out.at[c, pl.ds(r0, 64)])
        return 0

    lax.fori_loop(0, ROWS_PER_TILE // 64, cu, 0)
    pltpu.sync_copy(den_v, den_out.at[c, s])


_sc_edge = pl.kernel(
    _sc_edge_body,
    out_type=(
        jax.ShapeDtypeStruct((NC, NPAD, DH), jnp.float32),
        jax.ShapeDtypeStruct((NC, NS, NPAD), jnp.float32),
    ),
    mesh=plsc.VectorSubcoreMesh(
        core_axis_name="c", subcore_axis_name="s", num_cores=NC, num_subcores=NS
    ),
    compiler_params=pltpu.CompilerParams(
        needs_layout_passes=False, use_tc_tiling_on_sc=False
    ),
    scratch_types=[
        pltpu.VMEM((NGRP, G), jnp.int32),
        pltpu.VMEM((NGRP, G), jnp.int32),
        pltpu.VMEM((NPAD,), jnp.float32),
        pltpu.VMEM((NPAD,), jnp.float32),
        pltpu.VMEM((NPAD,), jnp.float32),
        pltpu.VMEM((G,), jnp.float32),
        pltpu.VMEM((G,), jnp.float32),
        pltpu.VMEM((G,), jnp.float32),
        pltpu.VMEM((G, DH), jnp.float32),
        pltpu.VMEM((G, DH), jnp.float32),
        pltpu.VMEM((G, DH), jnp.float32),
        pltpu.VMEM_SHARED((NPAD, DH), jnp.float32),
        pltpu.SemaphoreType.DMA,
        pltpu.SemaphoreType.DMA,
        pltpu.SemaphoreType.DMA,
        pltpu.SemaphoreType.DMA,
        pltpu.SemaphoreType.DMA,
        pltpu.SemaphoreType.DMA,
    ],
)


def kernel(x, edge_index, batch, W1, a1_src, a1_dst, b1, W2, a2_src, a2_dst, b2):
    loop = jnp.arange(N, dtype=jnp.int32)
    pad = E_PAD - E_TOT
    src = jnp.concatenate([edge_index[0], loop, jnp.zeros((pad,), jnp.int32)])
    dst = jnp.concatenate([edge_index[1], loop, jnp.full((pad,), N, jnp.int32)])
    src3 = src.reshape(NS, NGRP, G)
    dst3 = dst.reshape(NS, NGRP, G)

    x_pad = jnp.concatenate([x, jnp.zeros((NPAD - N, D), jnp.float32)])
    A1 = jnp.zeros((D, 8), jnp.float32).at[:, 0].set(a1_src).at[:, 1].set(a1_dst)
    A2 = jnp.zeros((D, 8), jnp.float32).at[:, 0].set(a2_src).at[:, 1].set(a2_dst)
    b1r = jnp.zeros((8, D), jnp.float32).at[0].set(b1)
    b2r = jnp.zeros((8, D), jnp.float32).at[0].set(b2)
    batch3 = jnp.concatenate(
        [batch.astype(jnp.int32), jnp.full((NPAD - N,), NB, jnp.int32)]
    ).reshape(GRID, 1, RB)

    h1, sc1 = _tc_prep(x_pad, W1, A1)
    U1, den1 = _sc_edge(src3, dst3, sc1[:, 0], sc1[:, 1], h1)
    h2, sc2 = _tc_mid(U1, den1, b1r, W2, A2)
    U2, den2 = _sc_edge(src3, dst3, sc2[:, 0], sc2[:, 1], h2)
    return _tc_pool(U2, den2, b2r, batch3)
